# Initial kernel scaffold; baseline (speedup 1.0000x reference)
import jax
import jax.numpy as jnp
from jax.experimental import pallas as pl


def kernel(points, dw, weight, bias):
    raise NotImplementedError("write your pallas kernel here")


# speedup vs baseline: 6.1645x; 6.1645x over previous
import functools

import jax
import jax.numpy as jnp
from jax import lax
from jax.experimental import pallas as pl
from jax.experimental.pallas import tpu as pltpu
from jax.experimental.pallas import tpu_sc as plsc

_N = 8192
_K = 81
_BLK = 256
_D = 16
_KD = _K * _D


def _topk_body(xnb_ref, xnt_ref, idx_ref, sim_ref):
    n = xnt_ref.shape[1]
    b = xnb_ref.shape[0]
    row0 = pl.program_id(0) * b
    col_ids = lax.broadcasted_iota(jnp.int32, (b, n), 1)
    row_ids = row0 + lax.broadcasted_iota(jnp.int32, (b, n), 0)
    xb16 = xnb_ref[:, :].astype(jnp.bfloat16)
    yt16 = xnt_ref[:, :].astype(jnp.bfloat16)
    sim = jnp.dot(xb16, yt16, preferred_element_type=jnp.float32)
    sim = jnp.where(col_ids == row_ids, jnp.float32(2.0), sim)
    sim_ref[:, :] = sim

    lane_k = lax.broadcasted_iota(jnp.int32, (b, _K), 1)

    def body(j, _):
        s = sim_ref[:, :]
        m = jnp.max(s, axis=1, keepdims=True)
        cand = jnp.where(s == m, col_ids, jnp.int32(n))
        a = jnp.min(cand, axis=1, keepdims=True)
        sim_ref[:, :] = jnp.where(col_ids == a, jnp.float32(-3.0), s)
        idx_ref[:, :] = jnp.where(lane_k == j, a, idx_ref[:, :])
        return 0

    idx_ref[:, :] = jnp.zeros((b, _K), jnp.int32)
    lax.fori_loop(0, _K, body, 0)


def _topk_call(xn, xnt):
    n = xn.shape[0]
    grid = n // _BLK
    return pl.pallas_call(
        _topk_body,
        grid=(grid,),
        in_specs=[
            pl.BlockSpec((_BLK, 3), lambda i: (i, 0)),
            pl.BlockSpec((3, n), lambda i: (0, 0)),
        ],
        out_specs=pl.BlockSpec((_BLK, _K), lambda i: (i, 0)),
        out_shape=jax.ShapeDtypeStruct((n, _K), jnp.int32),
        scratch_shapes=[pltpu.VMEM((_BLK, n), jnp.float32)],
    )(xn, xnt)


def _make_sc_gather(n, b_total):
    info = plsc.get_sparse_core_info()
    nw = info.num_cores * info.num_subcores
    b_per_w = b_total // nw
    steps = b_per_w // 16
    mesh = plsc.VectorSubcoreMesh(core_axis_name="c", subcore_axis_name="s")
    f32 = jnp.float32

    @functools.partial(
        pl.kernel,
        mesh=mesh,
        out_type=(jax.ShapeDtypeStruct((b_total,), f32),) * 3,
        compiler_params=pltpu.CompilerParams(needs_layout_passes=False),
        scratch_types=[
            pltpu.VMEM((n,), f32),
            pltpu.VMEM((n,), f32),
            pltpu.VMEM((n,), f32),
            pltpu.VMEM((b_per_w,), jnp.int32),
            pltpu.VMEM((b_per_w,), f32),
            pltpu.VMEM((b_per_w,), f32),
            pltpu.VMEM((b_per_w,), f32),
        ],
    )
    def sc_gather(x_hbm, y_hbm, z_hbm, idx_hbm, ox_hbm, oy_hbm, oz_hbm,
                  xv, yv, zv, idx_v, gx_v, gy_v, gz_v):
        wid = lax.axis_index("s") * info.num_cores + lax.axis_index("c")
        base = wid * b_per_w
        pltpu.sync_copy(x_hbm, xv)
        pltpu.sync_copy(y_hbm, yv)
        pltpu.sync_copy(z_hbm, zv)
        pltpu.sync_copy(idx_hbm.at[pl.ds(base, b_per_w)], idx_v)

        def body(i, _):
            o = i * 16
            iv = idx_v[pl.ds(o, 16)]
            gx_v[pl.ds(o, 16)] = plsc.load_gather(xv, [iv])
            gy_v[pl.ds(o, 16)] = plsc.load_gather(yv, [iv])
            gz_v[pl.ds(o, 16)] = plsc.load_gather(zv, [iv])
            return 0

        lax.fori_loop(0, steps, body, 0)
        pltpu.sync_copy(gx_v, ox_hbm.at[pl.ds(base, b_per_w)])
        pltpu.sync_copy(gy_v, oy_hbm.at[pl.ds(base, b_per_w)])
        pltpu.sync_copy(gz_v, oz_hbm.at[pl.ds(base, b_per_w)])

    return sc_gather


def _assemble_body(sx_ref, sy_ref, sz_ref, pts_ref, dwt_ref, w_ref,
                   bias_ref, out_ref):
    f32 = jnp.float32
    gx = sx_ref[:, :]
    gy = sy_ref[:, :]
    gz = sz_ref[:, :]
    dx = gx - pts_ref[:, 0:1]
    dy = gy - pts_ref[:, 1:2]
    dz = gz - pts_ref[:, 2:3]
    dist = dx * dx + dy * dy + dz * dz + jnp.float32(1.0)
    acc = jnp.dot(gx * dwt_ref[0:1, :], w_ref[0], preferred_element_type=f32)
    acc = acc + jnp.dot(gy * dwt_ref[1:2, :], w_ref[1],
                        preferred_element_type=f32)
    acc = acc + jnp.dot(gz * dwt_ref[2:3, :], w_ref[2],
                        preferred_element_type=f32)
    acc = acc + jnp.dot(dist * dwt_ref[3:4, :], w_ref[3],
                        preferred_element_type=f32)
    out_ref[:, :] = acc + bias_ref[:, :]


def _assemble_call(selx, sely, selz, points, dwt, weight, bias2d):
    n = points.shape[0]
    grid = n // _BLK
    cout = weight.shape[2]
    return pl.pallas_call(
        _assemble_body,
        grid=(grid,),
        in_specs=[
            pl.BlockSpec((_BLK, _K), lambda i: (i, 0)),
            pl.BlockSpec((_BLK, _K), lambda i: (i, 0)),
            pl.BlockSpec((_BLK, _K), lambda i: (i, 0)),
            pl.BlockSpec((_BLK, 3), lambda i: (i, 0)),
            pl.BlockSpec((4, _K), lambda i: (0, 0)),
            pl.BlockSpec((4, _K, cout), lambda i: (0, 0, 0)),
            pl.BlockSpec((1, cout), lambda i: (0, 0)),
        ],
        out_specs=pl.BlockSpec((_BLK, cout), lambda i: (i, 0)),
        out_shape=jax.ShapeDtypeStruct((n, cout), jnp.float32),
    )(selx, sely, selz, points, dwt, weight, bias2d)


def kernel(points, dw, weight, bias):
    n = points.shape[0]
    cout = weight.shape[2]
    norm = jnp.linalg.norm(points[:, :3], axis=-1, keepdims=True)
    xn = points[:, :3] / jnp.maximum(norm, 1e-12)
    xnt = xn.T

    idx = _topk_call(xn, xnt)

    idx_flat = idx.reshape(-1)
    px = jnp.asarray(points[:, 0], jnp.float32)
    py = jnp.asarray(points[:, 1], jnp.float32)
    pz = jnp.asarray(points[:, 2], jnp.float32)
    sx, sy, sz = _make_sc_gather(n, n * _K)(px, py, pz, idx_flat)
    selx = sx.reshape(n, _K)
    sely = sy.reshape(n, _K)
    selz = sz.reshape(n, _K)

    return _assemble_call(selx, sely, selz, points, dw.T, weight,
                          bias.reshape(1, cout))


# speedup vs baseline: 11.2782x; 1.8295x over previous
import functools

import jax
import jax.numpy as jnp
from jax import lax
from jax.experimental import pallas as pl
from jax.experimental.pallas import tpu as pltpu
from jax.experimental.pallas import tpu_sc as plsc

_N = 8192
_K = 81
_BLK = 256
_C = 128
_BIS = 14


def _sim_thr_body(xnb_ref, xnt_ref, sim_ref, thr_ref):
    b, n = sim_ref.shape
    row0 = pl.program_id(0) * b
    col_ids = lax.broadcasted_iota(jnp.int32, (b, n), 1)
    row_ids = row0 + lax.broadcasted_iota(jnp.int32, (b, n), 0)
    xb16 = xnb_ref[:, :].astype(jnp.bfloat16)
    yt16 = xnt_ref[:, :].astype(jnp.bfloat16)
    sim = jnp.dot(xb16, yt16, preferred_element_type=jnp.float32)
    sim = jnp.where(col_ids == row_ids, jnp.float32(2.0), sim)
    sim_ref[:, :] = sim

    lo0 = jnp.full((b, 1), -1.0, jnp.float32)
    hi0 = jnp.full((b, 1), 1.0, jnp.float32)

    def bis(_, c):
        lo, hi = c
        mid = jnp.float32(0.5) * (lo + hi)
        cnt = jnp.sum(jnp.where(sim_ref[:, :] >= mid, 1.0, 0.0),
                      axis=1, keepdims=True)
        p = cnt >= jnp.float32(_K)
        return (jnp.where(p, mid, lo), jnp.where(p, hi, mid))

    lo, hi = lax.fori_loop(0, _BIS, bis, (lo0, hi0))
    thr_ref[:, :] = jnp.broadcast_to(lo, (b, 16))


def _sim_thr_call(xn, xnt):
    n = xn.shape[0]
    grid = n // _BLK
    return pl.pallas_call(
        _sim_thr_body,
        grid=(grid,),
        in_specs=[
            pl.BlockSpec((_BLK, 3), lambda i: (i, 0)),
            pl.BlockSpec((3, n), lambda i: (0, 0)),
        ],
        out_specs=[
            pl.BlockSpec((_BLK, n), lambda i: (i, 0)),
            pl.BlockSpec((_BLK, 16), lambda i: (i, 0)),
        ],
        out_shape=[
            jax.ShapeDtypeStruct((n, n), jnp.float32),
            jax.ShapeDtypeStruct((n, 16), jnp.float32),
        ],
    )(xn, xnt)


def _make_sc_compact(n):
    info = plsc.get_sparse_core_info()
    nw = info.num_cores * info.num_subcores
    rows_w = n // nw
    rb_rows = 64
    nvec = n // 16
    i32, f32 = jnp.int32, jnp.float32
    mesh = plsc.VectorSubcoreMesh(core_axis_name="c", subcore_axis_name="s")

    @functools.partial(
        pl.kernel,
        mesh=mesh,
        out_type=(jax.ShapeDtypeStruct((n * _C,), f32),) * 4,
        compiler_params=pltpu.CompilerParams(needs_layout_passes=False),
        scratch_types=[
            pltpu.VMEM((n,), f32),
            pltpu.VMEM((n,), f32),
            pltpu.VMEM((n,), f32),
            pltpu.VMEM((n,), f32),
            pltpu.VMEM((rows_w * 16,), f32),
            pltpu.VMEM((144,), i32),
            pltpu.VMEM((rb_rows * _C,), f32),
            pltpu.VMEM((rb_rows * _C,), f32),
            pltpu.VMEM((rb_rows * _C,), f32),
            pltpu.VMEM((rb_rows * _C,), f32),
        ],
    )
    def sc_compact(sim_hbm, thr_hbm, x_hbm, y_hbm, z_hbm,
                   cv_hbm, cx_hbm, cy_hbm, cz_hbm,
                   xv, yv, zv, rowbuf, thrv, ci, ov, ox, oy, oz):
        wid = lax.axis_index("s") * info.num_cores + lax.axis_index("c")
        r0 = wid * rows_w
        pltpu.sync_copy(x_hbm, xv)
        pltpu.sync_copy(y_hbm, yv)
        pltpu.sync_copy(z_hbm, zv)
        pltpu.sync_copy(thr_hbm.at[pl.ds(r0 * 16, rows_w * 16)], thrv)
        iota16 = lax.iota(i32, 16)

        for bb in range(rows_w // rb_rows):
            def row_body(r, _, bb=bb):
                g = r0 + bb * rb_rows + r
                pltpu.sync_copy(sim_hbm.at[pl.ds(g * n, n)], rowbuf)
                tv = thrv[pl.ds((bb * rb_rows + r) * 16, 16)]

                def scan(vb, ptr):
                    o = vb * 16
                    s = rowbuf[pl.ds(o, 16)]
                    msk = s >= tv
                    iv = iota16 + o
                    plsc.store_compressed(
                        ci.at[pl.ds(jnp.minimum(ptr, _C), 16)], iv, mask=msk)
                    return ptr + jnp.sum(msk.astype(i32))

                cnt = lax.fori_loop(0, nvec, scan, jnp.int32(0))
                cnt16 = jnp.zeros((16,), i32) + cnt
                for t in range(_C // 16):
                    ii = ci[pl.ds(t * 16, 16)]
                    valid = (iota16 + t * 16) < cnt16
                    vals = plsc.load_gather(rowbuf, [ii], mask=valid)
                    ob = r * _C + t * 16
                    ov[pl.ds(ob, 16)] = jnp.where(
                        valid, vals, jnp.float32(-3.0))
                    ox[pl.ds(ob, 16)] = plsc.load_gather(xv, [ii], mask=valid)
                    oy[pl.ds(ob, 16)] = plsc.load_gather(yv, [ii], mask=valid)
                    oz[pl.ds(ob, 16)] = plsc.load_gather(zv, [ii], mask=valid)
                return 0

            lax.fori_loop(0, rb_rows, row_body, 0)
            base = (r0 + bb * rb_rows) * _C
            pltpu.sync_copy(ov, cv_hbm.at[pl.ds(base, rb_rows * _C)])
            pltpu.sync_copy(ox, cx_hbm.at[pl.ds(base, rb_rows * _C)])
            pltpu.sync_copy(oy, cy_hbm.at[pl.ds(base, rb_rows * _C)])
            pltpu.sync_copy(oz, cz_hbm.at[pl.ds(base, rb_rows * _C)])

    return sc_compact


def _rank_body(cv_ref, cx_ref, cy_ref, cz_ref, pts_ref, dwt_ref, w_ref,
               bias_ref, out_ref, cv_s, sx_s, sy_s, sz_s):
    f32 = jnp.float32
    b = cv_ref.shape[0]
    lane_c = lax.broadcasted_iota(jnp.int32, (b, _C), 1)
    lane_k = lax.broadcasted_iota(jnp.int32, (b, _K), 1)
    cv_s[:, :] = cv_ref[:, :]
    cx = cx_ref[:, :]
    cy = cy_ref[:, :]
    cz = cz_ref[:, :]
    sx_s[:, :] = jnp.zeros((b, _K), f32)
    sy_s[:, :] = jnp.zeros((b, _K), f32)
    sz_s[:, :] = jnp.zeros((b, _K), f32)

    def body(j, _):
        cv = cv_s[:, :]
        m = jnp.max(cv, axis=1, keepdims=True)
        a = jnp.min(jnp.where(cv == m, lane_c, jnp.int32(_C)),
                    axis=1, keepdims=True)
        oh = lane_c == a
        vx = jnp.sum(jnp.where(oh, cx, 0.0), axis=1, keepdims=True)
        vy = jnp.sum(jnp.where(oh, cy, 0.0), axis=1, keepdims=True)
        vz = jnp.sum(jnp.where(oh, cz, 0.0), axis=1, keepdims=True)
        cv_s[:, :] = jnp.where(oh, jnp.float32(-3.0), cv)
        kj = lane_k == j
        sx_s[:, :] = jnp.where(kj, vx, sx_s[:, :])
        sy_s[:, :] = jnp.where(kj, vy, sy_s[:, :])
        sz_s[:, :] = jnp.where(kj, vz, sz_s[:, :])
        return 0

    lax.fori_loop(0, _K, body, 0)

    gx = sx_s[:, :]
    gy = sy_s[:, :]
    gz = sz_s[:, :]
    dx = gx - pts_ref[:, 0:1]
    dy = gy - pts_ref[:, 1:2]
    dz = gz - pts_ref[:, 2:3]
    dist = dx * dx + dy * dy + dz * dz + jnp.float32(1.0)
    acc = jnp.dot(gx * dwt_ref[0:1, :], w_ref[0], preferred_element_type=f32)
    acc = acc + jnp.dot(gy * dwt_ref[1:2, :], w_ref[1],
                        preferred_element_type=f32)
    acc = acc + jnp.dot(gz * dwt_ref[2:3, :], w_ref[2],
                        preferred_element_type=f32)
    acc = acc + jnp.dot(dist * dwt_ref[3:4, :], w_ref[3],
                        preferred_element_type=f32)
    out_ref[:, :] = acc + bias_ref[:, :]


def _rank_call(cv, cx, cy, cz, points, dwt, weight, bias2d):
    n = points.shape[0]
    grid = n // _BLK
    cout = weight.shape[2]
    return pl.pallas_call(
        _rank_body,
        grid=(grid,),
        in_specs=[
            pl.BlockSpec((_BLK, _C), lambda i: (i, 0)),
            pl.BlockSpec((_BLK, _C), lambda i: (i, 0)),
            pl.BlockSpec((_BLK, _C), lambda i: (i, 0)),
            pl.BlockSpec((_BLK, _C), lambda i: (i, 0)),
            pl.BlockSpec((_BLK, 3), lambda i: (i, 0)),
            pl.BlockSpec((4, _K), lambda i: (0, 0)),
            pl.BlockSpec((4, _K, cout), lambda i: (0, 0, 0)),
            pl.BlockSpec((1, cout), lambda i: (0, 0)),
        ],
        out_specs=pl.BlockSpec((_BLK, cout), lambda i: (i, 0)),
        out_shape=jax.ShapeDtypeStruct((n, cout), jnp.float32),
        scratch_shapes=[
            pltpu.VMEM((_BLK, _C), jnp.float32),
            pltpu.VMEM((_BLK, _K), jnp.float32),
            pltpu.VMEM((_BLK, _K), jnp.float32),
            pltpu.VMEM((_BLK, _K), jnp.float32),
        ],
    )(cv, cx, cy, cz, points, dwt, weight, bias2d)


def kernel(points, dw, weight, bias):
    n = points.shape[0]
    cout = weight.shape[2]
    norm = jnp.linalg.norm(points[:, :3], axis=-1, keepdims=True)
    xn = points[:, :3] / jnp.maximum(norm, 1e-12)
    xnt = xn.T

    sim, thr16 = _sim_thr_call(xn, xnt)

    px = jnp.asarray(points[:, 0], jnp.float32)
    py = jnp.asarray(points[:, 1], jnp.float32)
    pz = jnp.asarray(points[:, 2], jnp.float32)
    cv1, cx1, cy1, cz1 = _make_sc_compact(n)(
        sim.reshape(-1), thr16.reshape(-1), px, py, pz)
    cv = cv1.reshape(n, _C)
    cx = cx1.reshape(n, _C)
    cy = cy1.reshape(n, _C)
    cz = cz1.reshape(n, _C)

    return _rank_call(cv, cx, cy, cz, points, dw.T, weight,
                      bias.reshape(1, cout))


# speedup vs baseline: 11.5800x; 1.0268x over previous
import functools

import jax
import jax.numpy as jnp
from jax import lax
from jax.experimental import pallas as pl
from jax.experimental.pallas import tpu as pltpu
from jax.experimental.pallas import tpu_sc as plsc

_N = 8192
_K = 81
_BLK = 256
_C = 128
_BIS = 12


def _sim_thr_body(xnb_ref, xnt_ref, sim_ref, thr_ref):
    b, n = sim_ref.shape
    row0 = pl.program_id(0) * b
    col_ids = lax.broadcasted_iota(jnp.int32, (b, n), 1)
    row_ids = row0 + lax.broadcasted_iota(jnp.int32, (b, n), 0)
    xb16 = xnb_ref[:, :].astype(jnp.bfloat16)
    yt16 = xnt_ref[:, :].astype(jnp.bfloat16)
    sim = jnp.dot(xb16, yt16, preferred_element_type=jnp.float32)
    sim = jnp.where(col_ids == row_ids, jnp.float32(2.0), sim)
    sim_ref[:, :] = sim

    lo0 = jnp.full((b, 1), -1.0, jnp.float32)
    hi0 = jnp.full((b, 1), 1.0, jnp.float32)

    def bis(_, c):
        lo, hi = c
        mid = jnp.float32(0.5) * (lo + hi)
        cnt = jnp.sum(jnp.where(sim_ref[:, :] >= mid, 1.0, 0.0),
                      axis=1, keepdims=True)
        p = cnt >= jnp.float32(_K)
        return (jnp.where(p, mid, lo), jnp.where(p, hi, mid))

    lo, hi = lax.fori_loop(0, _BIS, bis, (lo0, hi0))
    thr_ref[:, :] = jnp.broadcast_to(lo, (b, 16))


def _sim_thr_call(xn, xnt):
    n = xn.shape[0]
    grid = n // _BLK
    return pl.pallas_call(
        _sim_thr_body,
        grid=(grid,),
        in_specs=[
            pl.BlockSpec((_BLK, 3), lambda i: (i, 0)),
            pl.BlockSpec((3, n), lambda i: (0, 0)),
        ],
        out_specs=[
            pl.BlockSpec((_BLK, n), lambda i: (i, 0)),
            pl.BlockSpec((_BLK, 16), lambda i: (i, 0)),
        ],
        out_shape=[
            jax.ShapeDtypeStruct((n, n), jnp.float32),
            jax.ShapeDtypeStruct((n, 16), jnp.float32),
        ],
    )(xn, xnt)


def _make_sc_compact(n):
    info = plsc.get_sparse_core_info()
    nw = info.num_cores * info.num_subcores
    rows_w = n // nw
    rb_rows = 64
    nvec = n // 16
    i32, f32 = jnp.int32, jnp.float32
    mesh = plsc.VectorSubcoreMesh(core_axis_name="c", subcore_axis_name="s")

    @functools.partial(
        pl.kernel,
        mesh=mesh,
        out_type=(jax.ShapeDtypeStruct((n * _C,), f32),) * 4,
        compiler_params=pltpu.CompilerParams(needs_layout_passes=False),
        scratch_types=[
            pltpu.VMEM((n,), f32),
            pltpu.VMEM((n,), f32),
            pltpu.VMEM((n,), f32),
            pltpu.VMEM((n,), f32),
            pltpu.VMEM((rows_w * 16,), f32),
            pltpu.VMEM((144,), i32),
            pltpu.VMEM((rb_rows * _C,), f32),
            pltpu.VMEM((rb_rows * _C,), f32),
            pltpu.VMEM((rb_rows * _C,), f32),
            pltpu.VMEM((rb_rows * _C,), f32),
        ],
    )
    def sc_compact(sim_hbm, thr_hbm, x_hbm, y_hbm, z_hbm,
                   cv_hbm, cx_hbm, cy_hbm, cz_hbm,
                   xv, yv, zv, rowbuf, thrv, ci, ov, ox, oy, oz):
        wid = lax.axis_index("s") * info.num_cores + lax.axis_index("c")
        r0 = wid * rows_w
        pltpu.sync_copy(x_hbm, xv)
        pltpu.sync_copy(y_hbm, yv)
        pltpu.sync_copy(z_hbm, zv)
        pltpu.sync_copy(thr_hbm.at[pl.ds(r0 * 16, rows_w * 16)], thrv)
        iota16 = lax.iota(i32, 16)

        for bb in range(rows_w // rb_rows):
            def row_body(r, _, bb=bb):
                g = r0 + bb * rb_rows + r
                pltpu.sync_copy(sim_hbm.at[pl.ds(g * n, n)], rowbuf)
                tv = thrv[pl.ds((bb * rb_rows + r) * 16, 16)]

                def scan(vb, ptr):
                    o = vb * 16
                    s = rowbuf[pl.ds(o, 16)]
                    msk = s >= tv
                    iv = iota16 + o
                    plsc.store_compressed(
                        ci.at[pl.ds(jnp.minimum(ptr, _C), 16)], iv, mask=msk)
                    c16 = plsc.all_reduce_population_count(msk)
                    return ptr + c16[0]

                cnt = lax.fori_loop(0, nvec, scan, jnp.int32(0), unroll=8)
                cnt16 = jnp.zeros((16,), i32) + cnt
                for t in range(_C // 16):
                    ii = ci[pl.ds(t * 16, 16)]
                    valid = (iota16 + t * 16) < cnt16
                    vals = plsc.load_gather(rowbuf, [ii], mask=valid)
                    ob = r * _C + t * 16
                    ov[pl.ds(ob, 16)] = jnp.where(
                        valid, vals, jnp.float32(-3.0))
                    ox[pl.ds(ob, 16)] = plsc.load_gather(xv, [ii], mask=valid)
                    oy[pl.ds(ob, 16)] = plsc.load_gather(yv, [ii], mask=valid)
                    oz[pl.ds(ob, 16)] = plsc.load_gather(zv, [ii], mask=valid)
                return 0

            lax.fori_loop(0, rb_rows, row_body, 0)
            base = (r0 + bb * rb_rows) * _C
            pltpu.sync_copy(ov, cv_hbm.at[pl.ds(base, rb_rows * _C)])
            pltpu.sync_copy(ox, cx_hbm.at[pl.ds(base, rb_rows * _C)])
            pltpu.sync_copy(oy, cy_hbm.at[pl.ds(base, rb_rows * _C)])
            pltpu.sync_copy(oz, cz_hbm.at[pl.ds(base, rb_rows * _C)])

    return sc_compact


def _rank_body(cv_ref, cx_ref, cy_ref, cz_ref, pts_ref, dwt_ref, w_ref,
               bias_ref, out_ref, cv_s, sx_s, sy_s, sz_s):
    f32 = jnp.float32
    b = cv_ref.shape[0]
    lane_c = lax.broadcasted_iota(jnp.int32, (b, _C), 1)
    lane_k = lax.broadcasted_iota(jnp.int32, (b, _K), 1)
    cv_s[:, :] = cv_ref[:, :]
    cx = cx_ref[:, :]
    cy = cy_ref[:, :]
    cz = cz_ref[:, :]
    sx_s[:, :] = jnp.zeros((b, _K), f32)
    sy_s[:, :] = jnp.zeros((b, _K), f32)
    sz_s[:, :] = jnp.zeros((b, _K), f32)

    def body(j, _):
        cv = cv_s[:, :]
        m = jnp.max(cv, axis=1, keepdims=True)
        a = jnp.min(jnp.where(cv == m, lane_c, jnp.int32(_C)),
                    axis=1, keepdims=True)
        oh = lane_c == a
        vx = jnp.sum(jnp.where(oh, cx, 0.0), axis=1, keepdims=True)
        vy = jnp.sum(jnp.where(oh, cy, 0.0), axis=1, keepdims=True)
        vz = jnp.sum(jnp.where(oh, cz, 0.0), axis=1, keepdims=True)
        cv_s[:, :] = jnp.where(oh, jnp.float32(-3.0), cv)
        kj = lane_k == j
        sx_s[:, :] = jnp.where(kj, vx, sx_s[:, :])
        sy_s[:, :] = jnp.where(kj, vy, sy_s[:, :])
        sz_s[:, :] = jnp.where(kj, vz, sz_s[:, :])
        return 0

    lax.fori_loop(0, _K, body, 0)

    gx = sx_s[:, :]
    gy = sy_s[:, :]
    gz = sz_s[:, :]
    dx = gx - pts_ref[:, 0:1]
    dy = gy - pts_ref[:, 1:2]
    dz = gz - pts_ref[:, 2:3]
    dist = dx * dx + dy * dy + dz * dz + jnp.float32(1.0)
    acc = jnp.dot(gx * dwt_ref[0:1, :], w_ref[0], preferred_element_type=f32)
    acc = acc + jnp.dot(gy * dwt_ref[1:2, :], w_ref[1],
                        preferred_element_type=f32)
    acc = acc + jnp.dot(gz * dwt_ref[2:3, :], w_ref[2],
                        preferred_element_type=f32)
    acc = acc + jnp.dot(dist * dwt_ref[3:4, :], w_ref[3],
                        preferred_element_type=f32)
    out_ref[:, :] = acc + bias_ref[:, :]


def _rank_call(cv, cx, cy, cz, points, dwt, weight, bias2d):
    n = points.shape[0]
    grid = n // _BLK
    cout = weight.shape[2]
    return pl.pallas_call(
        _rank_body,
        grid=(grid,),
        in_specs=[
            pl.BlockSpec((_BLK, _C), lambda i: (i, 0)),
            pl.BlockSpec((_BLK, _C), lambda i: (i, 0)),
            pl.BlockSpec((_BLK, _C), lambda i: (i, 0)),
            pl.BlockSpec((_BLK, _C), lambda i: (i, 0)),
            pl.BlockSpec((_BLK, 3), lambda i: (i, 0)),
            pl.BlockSpec((4, _K), lambda i: (0, 0)),
            pl.BlockSpec((4, _K, cout), lambda i: (0, 0, 0)),
            pl.BlockSpec((1, cout), lambda i: (0, 0)),
        ],
        out_specs=pl.BlockSpec((_BLK, cout), lambda i: (i, 0)),
        out_shape=jax.ShapeDtypeStruct((n, cout), jnp.float32),
        scratch_shapes=[
            pltpu.VMEM((_BLK, _C), jnp.float32),
            pltpu.VMEM((_BLK, _K), jnp.float32),
            pltpu.VMEM((_BLK, _K), jnp.float32),
            pltpu.VMEM((_BLK, _K), jnp.float32),
        ],
    )(cv, cx, cy, cz, points, dwt, weight, bias2d)


def kernel(points, dw, weight, bias):
    n = points.shape[0]
    cout = weight.shape[2]
    norm = jnp.linalg.norm(points[:, :3], axis=-1, keepdims=True)
    xn = points[:, :3] / jnp.maximum(norm, 1e-12)
    xnt = xn.T

    sim, thr16 = _sim_thr_call(xn, xnt)

    px = jnp.asarray(points[:, 0], jnp.float32)
    py = jnp.asarray(points[:, 1], jnp.float32)
    pz = jnp.asarray(points[:, 2], jnp.float32)
    cv1, cx1, cy1, cz1 = _make_sc_compact(n)(
        sim.reshape(-1), thr16.reshape(-1), px, py, pz)
    cv = cv1.reshape(n, _C)
    cx = cx1.reshape(n, _C)
    cy = cy1.reshape(n, _C)
    cz = cz1.reshape(n, _C)

    return _rank_call(cv, cx, cy, cz, points, dw.T, weight,
                      bias.reshape(1, cout))
